# Initial kernel scaffold; baseline (speedup 1.0000x reference)
import jax
import jax.numpy as jnp
from jax.experimental import pallas as pl


def kernel(z, codebook):
    raise NotImplementedError("write your pallas kernel here")


# speedup vs baseline: 1.0570x; 1.0570x over previous
import functools

import jax
import jax.numpy as jnp
from jax import lax
from jax.experimental import pallas as pl
from jax.experimental.pallas import tpu as pltpu
from jax.experimental.pallas import tpu_sc as plsc


def _ids_body(z_ref, cb_ref, ids3_ref, ids_ref):
    zb = z_ref[0]
    cb = cb_ref[...]
    score = lax.dot_general(
        zb, cb, (((1,), (1,)), ((), ())),
        preferred_element_type=jnp.float32)
    score = score - 0.5 * jnp.sum(cb * cb, axis=1)[None, :]
    ids = jnp.argmax(score, axis=1).astype(jnp.int32)
    ids3_ref[0] = ids.reshape(ids3_ref.shape[1:])
    ids_ref[0, 0] = ids


def _tc_ids(z, codebook):
    B, T, D = z.shape
    K = codebook.shape[0]
    n_ch, CH = T // 128, 128
    ids3, ids = pl.pallas_call(
        _ids_body,
        grid=(B,),
        in_specs=[
            pl.BlockSpec((1, T, D), lambda i: (i, 0, 0)),
            pl.BlockSpec((K, D), lambda i: (0, 0)),
        ],
        out_specs=[
            pl.BlockSpec((1, n_ch, CH), lambda i: (i, 0, 0)),
            pl.BlockSpec((1, 1, T), lambda i: (i, 0, 0)),
        ],
        out_shape=[
            jax.ShapeDtypeStruct((B, n_ch, CH), jnp.int32),
            jax.ShapeDtypeStruct((B, 1, T), jnp.int32),
        ],
    )(z, codebook)
    return ids3, ids.reshape(B, T)


def _sc_gather(cb_padded, ids3, B, T):
    NW, n_ch, CH = ids3.shape
    mesh = plsc.VectorSubcoreMesh(core_axis_name="c", subcore_axis_name="s")

    @functools.partial(
        pl.kernel,
        mesh=mesh,
        out_type=jax.ShapeDtypeStruct((B, T, 128), jnp.float32),
        scratch_types=[
            pltpu.VMEM((n_ch, CH), jnp.int32),
            pltpu.VMEM((2, CH, 128), jnp.float32),
            pltpu.SemaphoreType.DMA,
            pltpu.SemaphoreType.DMA,
        ],
    )
    def k(cb_hbm, idx_hbm, out_hbm, idx_v, rows_v, sem0, sem1):
        nc = lax.axis_size("c")
        wid = lax.axis_index("s") * nc + lax.axis_index("c")
        pltpu.sync_copy(idx_hbm.at[wid], idx_v)
        sems = (sem0, sem1)

        def start(j):
            return pltpu.async_copy(
                cb_hbm.at[idx_v.at[j]], rows_v.at[j % 2], sems[j % 2])

        cp = start(0)
        for j in range(n_ch):
            nxt = start(j + 1) if j + 1 < n_ch else None
            cp.wait()
            pltpu.sync_copy(
                rows_v.at[j % 2], out_hbm.at[wid, pl.ds(j * CH, CH), :])
            cp = nxt

    return k(cb_padded, ids3)


def _depad_body(wide_ref, out_ref):
    out_ref[0] = wide_ref[0, :, : out_ref.shape[2]]


def _tc_depad(wide, D):
    B, T, W = wide.shape
    return pl.pallas_call(
        _depad_body,
        grid=(B,),
        in_specs=[pl.BlockSpec((1, T, W), lambda i: (i, 0, 0))],
        out_specs=pl.BlockSpec((1, T, D), lambda i: (i, 0, 0)),
        out_shape=jax.ShapeDtypeStruct((B, T, D), jnp.float32),
    )(wide)


def kernel(z, codebook):
    B, T, D = z.shape
    ids3, ids = _tc_ids(z, codebook)
    cb_padded = jnp.pad(codebook, ((0, 0), (0, 128 - D)))
    wide = _sc_gather(cb_padded, ids3, B, T)
    z_q = _tc_depad(wide, D)
    return z_q, ids


# speedup vs baseline: 2.2678x; 2.1456x over previous
import functools

import jax
import jax.numpy as jnp
from jax import lax
from jax.experimental import pallas as pl
from jax.experimental.pallas import tpu as pltpu
from jax.experimental.pallas import tpu_sc as plsc


def _ids_body(zt_ref, cbt_ref, ids3_ref, ids_ref):
    zbt = zt_ref[0]
    cbt = cbt_ref[...]
    score = lax.dot_general(
        cbt, zbt, (((0,), (0,)), ((), ())),
        preferred_element_type=jnp.float32)
    score = score - 0.5 * jnp.sum(cbt * cbt, axis=0)[:, None]
    ids = jnp.argmax(score, axis=0).astype(jnp.int32)
    ids3_ref[0] = ids.reshape(ids3_ref.shape[1:])
    ids_ref[0, 0] = ids


def _tc_ids(zt, cbt):
    B, D, T = zt.shape
    K = cbt.shape[1]
    n_ch, CH = T // 128, 128
    ids3, ids = pl.pallas_call(
        _ids_body,
        grid=(B,),
        in_specs=[
            pl.BlockSpec((1, D, T), lambda i: (i, 0, 0)),
            pl.BlockSpec((D, K), lambda i: (0, 0)),
        ],
        out_specs=[
            pl.BlockSpec((1, n_ch, CH), lambda i: (i, 0, 0)),
            pl.BlockSpec((1, 1, T), lambda i: (i, 0, 0)),
        ],
        out_shape=[
            jax.ShapeDtypeStruct((B, n_ch, CH), jnp.int32),
            jax.ShapeDtypeStruct((B, 1, T), jnp.int32),
        ],
    )(zt, cbt)
    return ids3, ids.reshape(B, T)


def _sc_gather_t(cbt, ids, B, D, T):
    K = cbt.shape[1]
    mesh = plsc.VectorSubcoreMesh(core_axis_name="c", subcore_axis_name="s")

    @functools.partial(
        pl.kernel,
        mesh=mesh,
        compiler_params=pltpu.CompilerParams(
            use_tc_tiling_on_sc=False, needs_layout_passes=False),
        out_type=jax.ShapeDtypeStruct((B, D, T), jnp.float32),
        scratch_types=[
            pltpu.VMEM((D, K), jnp.float32),
            pltpu.VMEM((T,), jnp.int32),
            pltpu.VMEM((D, T), jnp.float32),
        ],
    )
    def k(cb_hbm, idx_hbm, out_hbm, cb_v, idx_v, zq_v):
        nc = lax.axis_size("c")
        wid = lax.axis_index("s") * nc + lax.axis_index("c")
        pltpu.sync_copy(cb_hbm, cb_v)
        pltpu.sync_copy(idx_hbm.at[wid], idx_v)

        def band(i, carry):
            ids16 = idx_v[pl.ds(i * 16, 16)]
            for d in range(D):
                dvec = jnp.full((16,), d, jnp.int32)
                vals = plsc.load_gather(cb_v, [dvec, ids16])
                zq_v[d, pl.ds(i * 16, 16)] = vals
            return carry

        lax.fori_loop(0, T // 16, band, 0, unroll=False)
        pltpu.sync_copy(zq_v, out_hbm.at[wid])

    return k(cbt, ids)


def kernel(z, codebook):
    B, T, D = z.shape
    K = codebook.shape[0]
    zt = jnp.swapaxes(z, 1, 2)
    cbt = codebook.T
    ids3, ids = _tc_ids(zt, cbt)
    zq_t = _sc_gather_t(cbt, ids3.reshape(B, T), B, D, T)
    return jnp.swapaxes(zq_t, 1, 2), ids


# speedup vs baseline: 2.5552x; 1.1267x over previous
import functools

import jax
import jax.numpy as jnp
from jax import lax
from jax.experimental import pallas as pl
from jax.experimental.pallas import tpu as pltpu
from jax.experimental.pallas import tpu_sc as plsc


def _ids_body(zt_ref, cbt_ref, ids3_ref, ids_ref):
    zbt = zt_ref[0]
    cbt = cbt_ref[...]
    score = lax.dot_general(
        cbt, zbt, (((0,), (0,)), ((), ())),
        preferred_element_type=jnp.float32)
    score = score - 0.5 * jnp.sum(cbt * cbt, axis=0)[:, None]
    ids = jnp.argmax(score, axis=0).astype(jnp.int32)
    ids3_ref[0] = ids.reshape(ids3_ref.shape[1:])
    ids_ref[0, 0] = ids


def _tc_ids(zt, cbt):
    B, D, T = zt.shape
    K = cbt.shape[1]
    n_ch, CH = T // 128, 128
    ids3, ids = pl.pallas_call(
        _ids_body,
        grid=(B,),
        in_specs=[
            pl.BlockSpec((1, D, T), lambda i: (i, 0, 0)),
            pl.BlockSpec((D, K), lambda i: (0, 0)),
        ],
        out_specs=[
            pl.BlockSpec((1, n_ch, CH), lambda i: (i, 0, 0)),
            pl.BlockSpec((1, 1, T), lambda i: (i, 0, 0)),
        ],
        out_shape=[
            jax.ShapeDtypeStruct((B, n_ch, CH), jnp.int32),
            jax.ShapeDtypeStruct((B, 1, T), jnp.int32),
        ],
    )(zt, cbt)
    return ids3, ids.reshape(B, T)


def _sc_gather_t(cbt, ids, B, D, T):
    K = cbt.shape[1]
    mesh = plsc.VectorSubcoreMesh(core_axis_name="c", subcore_axis_name="s")

    @functools.partial(
        pl.kernel,
        mesh=mesh,
        compiler_params=pltpu.CompilerParams(
            use_tc_tiling_on_sc=False, needs_layout_passes=False),
        out_type=jax.ShapeDtypeStruct((B, D, T), jnp.float32),
        scratch_types=[
            pltpu.VMEM((D, K), jnp.float32),
            pltpu.VMEM((T,), jnp.int32),
            pltpu.VMEM((D, T), jnp.float32),
        ],
    )
    def k(cb_hbm, idx_hbm, out_hbm, cb_v, idx_v, zq_v):
        nc = lax.axis_size("c")
        wid = lax.axis_index("s") * nc + lax.axis_index("c")
        pltpu.sync_copy(cb_hbm, cb_v)
        pltpu.sync_copy(idx_hbm.at[wid], idx_v)

        @plsc.parallel_loop(0, T // 16, step=1, unroll=2)
        def band(i):
            ids16 = idx_v[pl.ds(i * 16, 16)]
            for d in range(D):
                dvec = jnp.full((16,), d, jnp.int32)
                vals = plsc.load_gather(cb_v, [dvec, ids16])
                zq_v[d, pl.ds(i * 16, 16)] = vals
        pltpu.sync_copy(zq_v, out_hbm.at[wid])

    return k(cbt, ids)


def kernel(z, codebook):
    B, T, D = z.shape
    K = codebook.shape[0]
    zt = jnp.swapaxes(z, 1, 2)
    cbt = codebook.T
    ids3, ids = _tc_ids(zt, cbt)
    zq_t = _sc_gather_t(cbt, ids3.reshape(B, T), B, D, T)
    return jnp.swapaxes(zq_t, 1, 2), ids


# speedup vs baseline: 3.0915x; 1.2099x over previous
import functools

import jax
import jax.numpy as jnp
from jax import lax
from jax.experimental import pallas as pl
from jax.experimental.pallas import tpu as pltpu
from jax.experimental.pallas import tpu_sc as plsc


def _ids_body(zt_ref, cbt_ref, ids3_ref, ids_ref):
    zbt = zt_ref[0]
    cbt = cbt_ref[...]
    score = lax.dot_general(
        cbt, zbt, (((0,), (0,)), ((), ())),
        preferred_element_type=jnp.float32)
    score = score - 0.5 * jnp.sum(cbt * cbt, axis=0)[:, None]
    ids = jnp.argmax(score, axis=0).astype(jnp.int32)
    ids3_ref[0] = ids.reshape(ids3_ref.shape[1:])
    ids_ref[0, 0] = ids


def _tc_ids(zt, cbt):
    B, D, T = zt.shape
    K = cbt.shape[1]
    n_ch, CH = T // 128, 128
    ids3, ids = pl.pallas_call(
        _ids_body,
        grid=(B,),
        in_specs=[
            pl.BlockSpec((1, D, T), lambda i: (i, 0, 0)),
            pl.BlockSpec((D, K), lambda i: (0, 0)),
        ],
        out_specs=[
            pl.BlockSpec((1, n_ch, CH), lambda i: (i, 0, 0)),
            pl.BlockSpec((1, 1, T), lambda i: (i, 0, 0)),
        ],
        out_shape=[
            jax.ShapeDtypeStruct((B, n_ch, CH), jnp.int32),
            jax.ShapeDtypeStruct((B, 1, T), jnp.int32),
        ],
    )(zt, cbt)
    return ids3, ids.reshape(B, T)


def _sc_gather_t(cbt, ids, B, D, T):
    K = cbt.shape[1]
    mesh = plsc.VectorSubcoreMesh(core_axis_name="c", subcore_axis_name="s")

    @functools.partial(
        pl.kernel,
        mesh=mesh,
        compiler_params=pltpu.CompilerParams(
            use_tc_tiling_on_sc=False, needs_layout_passes=False),
        out_type=jax.ShapeDtypeStruct((B, D // 8, T // 128, 8, 128),
                                      jnp.float32),
        scratch_types=[
            pltpu.VMEM((D, K), jnp.float32),
            pltpu.VMEM((T,), jnp.int32),
            pltpu.VMEM((D // 8, T // 128, 8, 128), jnp.float32),
        ],
    )
    def k(cb_hbm, idx_hbm, out_hbm, cb_v, idx_v, zq_v):
        nc = lax.axis_size("c")
        wid = lax.axis_index("s") * nc + lax.axis_index("c")
        pltpu.sync_copy(cb_hbm, cb_v)
        pltpu.sync_copy(idx_hbm.at[wid], idx_v)

        @plsc.parallel_loop(0, T // 16, step=1, unroll=4)
        def band(i):
            ids16 = idx_v[pl.ds(i * 16, 16)]
            tb = i // 8
            lo = (i % 8) * 16
            for d in range(D):
                dvec = jnp.full((16,), d, jnp.int32)
                vals = plsc.load_gather(cb_v, [dvec, ids16])
                zq_v[d // 8, tb, d % 8, pl.ds(lo, 16)] = vals
        pltpu.sync_copy(zq_v, out_hbm.at[wid])

    return k(cbt, ids)


def kernel(z, codebook):
    B, T, D = z.shape
    K = codebook.shape[0]
    zt = jnp.swapaxes(z, 1, 2)
    cbt = codebook.T
    ids3, ids = _tc_ids(zt, cbt)
    zq5 = _sc_gather_t(cbt, ids3.reshape(B, T), B, D, T)
    zq_t = zq5.transpose(0, 1, 3, 2, 4).reshape(B, D, T)
    return jnp.swapaxes(zq_t, 1, 2), ids


# speedup vs baseline: 3.0960x; 1.0015x over previous
import functools

import jax
import jax.numpy as jnp
from jax import lax
from jax.experimental import pallas as pl
from jax.experimental.pallas import tpu as pltpu
from jax.experimental.pallas import tpu_sc as plsc


def _ids_body(zt_ref, cbt_ref, ids3_ref, ids_ref):
    cbt = cbt_ref[...]
    bias = 0.5 * jnp.sum(cbt * cbt, axis=0)[:, None]
    T = zt_ref.shape[2]
    n_ch, CH = ids3_ref.shape[1], ids3_ref.shape[2]
    H = 2
    TH = T // H
    for h in range(H):
        zbt = zt_ref[0, :, h * TH:(h + 1) * TH]
        score = lax.dot_general(
            cbt, zbt, (((0,), (0,)), ((), ())),
            preferred_element_type=jnp.float32)
        score = score - bias
        ids = jnp.argmax(score, axis=0).astype(jnp.int32)
        ids3_ref[0, h * (n_ch // H):(h + 1) * (n_ch // H)] = (
            ids.reshape(n_ch // H, CH))
        ids_ref[0, 0, h * TH:(h + 1) * TH] = ids


def _tc_ids(zt, cbt):
    B, D, T = zt.shape
    K = cbt.shape[1]
    n_ch, CH = T // 128, 128
    ids3, ids = pl.pallas_call(
        _ids_body,
        grid=(B,),
        in_specs=[
            pl.BlockSpec((1, D, T), lambda i: (i, 0, 0)),
            pl.BlockSpec((D, K), lambda i: (0, 0)),
        ],
        out_specs=[
            pl.BlockSpec((1, n_ch, CH), lambda i: (i, 0, 0)),
            pl.BlockSpec((1, 1, T), lambda i: (i, 0, 0)),
        ],
        out_shape=[
            jax.ShapeDtypeStruct((B, n_ch, CH), jnp.int32),
            jax.ShapeDtypeStruct((B, 1, T), jnp.int32),
        ],
    )(zt, cbt)
    return ids3, ids.reshape(B, T)


def _sc_gather_t(cbt, ids, B, D, T):
    K = cbt.shape[1]
    mesh = plsc.VectorSubcoreMesh(core_axis_name="c", subcore_axis_name="s")

    @functools.partial(
        pl.kernel,
        mesh=mesh,
        compiler_params=pltpu.CompilerParams(
            use_tc_tiling_on_sc=False, needs_layout_passes=False),
        out_type=jax.ShapeDtypeStruct((B, D // 8, T // 128, 8, 128),
                                      jnp.float32),
        scratch_types=[
            pltpu.VMEM((D, K), jnp.float32),
            pltpu.VMEM((T,), jnp.int32),
            pltpu.VMEM((D // 8, T // 128, 8, 128), jnp.float32),
        ],
    )
    def k(cb_hbm, idx_hbm, out_hbm, cb_v, idx_v, zq_v):
        nc = lax.axis_size("c")
        wid = lax.axis_index("s") * nc + lax.axis_index("c")
        pltpu.sync_copy(cb_hbm, cb_v)
        pltpu.sync_copy(idx_hbm.at[wid], idx_v)

        @plsc.parallel_loop(0, T // 16, step=1, unroll=4)
        def band(i):
            ids16 = idx_v[pl.ds(i * 16, 16)]
            tb = i // 8
            lo = (i % 8) * 16
            for d in range(D):
                dvec = jnp.full((16,), d, jnp.int32)
                vals = plsc.load_gather(cb_v, [dvec, ids16])
                zq_v[d // 8, tb, d % 8, pl.ds(lo, 16)] = vals
        pltpu.sync_copy(zq_v, out_hbm.at[wid])

    return k(cbt, ids)


def kernel(z, codebook):
    B, T, D = z.shape
    K = codebook.shape[0]
    zt = jnp.swapaxes(z, 1, 2)
    cbt = codebook.T
    ids3, ids = _tc_ids(zt, cbt)
    zq5 = _sc_gather_t(cbt, ids3.reshape(B, T), B, D, T)
    zq_t = zq5.transpose(0, 1, 3, 2, 4).reshape(B, D, T)
    return jnp.swapaxes(zq_t, 1, 2), ids


# speedup vs baseline: 3.8944x; 1.2579x over previous
import functools

import jax
import jax.numpy as jnp
from jax import lax
from jax.experimental import pallas as pl
from jax.experimental.pallas import tpu as pltpu
from jax.experimental.pallas import tpu_sc as plsc


def _ids_body(zt_ref, cbt_ref, ids3_ref, ids_ref):
    cbt = cbt_ref[...]
    bias = 0.5 * jnp.sum(cbt * cbt, axis=0)[:, None]
    n_b = zt_ref.shape[0]
    n_ch, CH = ids3_ref.shape[2], ids3_ref.shape[3]
    for b in range(n_b):
        zbt = zt_ref[b]
        score = lax.dot_general(
            cbt, zbt, (((0,), (0,)), ((), ())),
            preferred_element_type=jnp.float32)
        score = score - bias
        ids = jnp.argmax(score, axis=0).astype(jnp.int32)
        ids3_ref[b, 0] = ids.reshape(n_ch, CH)
        ids_ref[b, 0] = ids


def _tc_ids(zt, cbt):
    B, D, T = zt.shape
    K = cbt.shape[1]
    n_ch, CH = T // 128, 128
    NB = 4
    ids3, ids = pl.pallas_call(
        _ids_body,
        grid=(B // NB,),
        in_specs=[
            pl.BlockSpec((NB, D, T), lambda i: (i, 0, 0)),
            pl.BlockSpec((D, K), lambda i: (0, 0)),
        ],
        out_specs=[
            pl.BlockSpec((NB, 1, n_ch, CH), lambda i: (i, 0, 0, 0)),
            pl.BlockSpec((NB, 1, T), lambda i: (i, 0, 0)),
        ],
        out_shape=[
            jax.ShapeDtypeStruct((B, 1, n_ch, CH), jnp.int32),
            jax.ShapeDtypeStruct((B, 1, T), jnp.int32),
        ],
    )(zt, cbt)
    return ids3.reshape(B, n_ch, CH), ids.reshape(B, T)


def _sc_gather_t(cbt, ids, B, D, T):
    K = cbt.shape[1]
    mesh = plsc.VectorSubcoreMesh(core_axis_name="c", subcore_axis_name="s")

    @functools.partial(
        pl.kernel,
        mesh=mesh,
        compiler_params=pltpu.CompilerParams(
            use_tc_tiling_on_sc=False, needs_layout_passes=False),
        out_type=jax.ShapeDtypeStruct((B, D // 8, T // 128, 8, 128),
                                      jnp.float32),
        scratch_types=[
            pltpu.VMEM((D, K), jnp.float32),
            pltpu.VMEM((T,), jnp.int32),
            pltpu.VMEM((D // 8, T // 128, 8, 128), jnp.float32),
        ],
    )
    def k(cb_hbm, idx_hbm, out_hbm, cb_v, idx_v, zq_v):
        nc = lax.axis_size("c")
        wid = lax.axis_index("s") * nc + lax.axis_index("c")
        pltpu.sync_copy(cb_hbm, cb_v)
        pltpu.sync_copy(idx_hbm.at[wid], idx_v)

        @plsc.parallel_loop(0, T // 16, step=1, unroll=4)
        def band(i):
            ids16 = idx_v[pl.ds(i * 16, 16)]
            tb = i // 8
            lo = (i % 8) * 16
            for d in range(D):
                dvec = jnp.full((16,), d, jnp.int32)
                vals = plsc.load_gather(cb_v, [dvec, ids16])
                zq_v[d // 8, tb, d % 8, pl.ds(lo, 16)] = vals
        pltpu.sync_copy(zq_v, out_hbm.at[wid])

    return k(cbt, ids)


def kernel(z, codebook):
    B, T, D = z.shape
    K = codebook.shape[0]
    zt = jnp.swapaxes(z, 1, 2)
    cbt = codebook.T
    ids3, ids = _tc_ids(zt, cbt)
    zq5 = _sc_gather_t(cbt, ids3.reshape(B, T), B, D, T)
    zq_t = zq5.transpose(0, 1, 3, 2, 4).reshape(B, D, T)
    return jnp.swapaxes(zq_t, 1, 2), ids


# speedup vs baseline: 3.9439x; 1.0127x over previous
import functools

import jax
import jax.numpy as jnp
from jax import lax
from jax.experimental import pallas as pl
from jax.experimental.pallas import tpu as pltpu
from jax.experimental.pallas import tpu_sc as plsc


def _ids_body(zt_ref, cbt_ref, ids3_ref, ids_ref):
    cbt = cbt_ref[...]
    bias = 0.5 * jnp.sum(cbt * cbt, axis=0)[:, None]
    n_b = zt_ref.shape[0]
    n_ch, CH = ids3_ref.shape[2], ids3_ref.shape[3]
    for b in range(n_b):
        zbt = zt_ref[b]
        score = lax.dot_general(
            cbt, zbt, (((0,), (0,)), ((), ())),
            preferred_element_type=jnp.float32)
        score = score - bias
        ids = jnp.argmax(score, axis=0).astype(jnp.int32)
        ids3_ref[b, 0] = ids.reshape(n_ch, CH)
        ids_ref[b, 0] = ids


def _tc_ids(zt, cbt):
    B, D, T = zt.shape
    K = cbt.shape[1]
    n_ch, CH = T // 128, 128
    NB = 8
    ids3, ids = pl.pallas_call(
        _ids_body,
        grid=(B // NB,),
        in_specs=[
            pl.BlockSpec((NB, D, T), lambda i: (i, 0, 0)),
            pl.BlockSpec((D, K), lambda i: (0, 0)),
        ],
        out_specs=[
            pl.BlockSpec((NB, 1, n_ch, CH), lambda i: (i, 0, 0, 0)),
            pl.BlockSpec((NB, 1, T), lambda i: (i, 0, 0)),
        ],
        out_shape=[
            jax.ShapeDtypeStruct((B, 1, n_ch, CH), jnp.int32),
            jax.ShapeDtypeStruct((B, 1, T), jnp.int32),
        ],
    )(zt, cbt)
    return ids3.reshape(B, n_ch, CH), ids.reshape(B, T)


def _sc_gather_t(cbt, ids, B, D, T):
    K = cbt.shape[1]
    mesh = plsc.VectorSubcoreMesh(core_axis_name="c", subcore_axis_name="s")

    @functools.partial(
        pl.kernel,
        mesh=mesh,
        compiler_params=pltpu.CompilerParams(
            use_tc_tiling_on_sc=False, needs_layout_passes=False),
        out_type=jax.ShapeDtypeStruct((B, D // 8, T // 128, 8, 128),
                                      jnp.float32),
        scratch_types=[
            pltpu.VMEM((D, K), jnp.float32),
            pltpu.VMEM((T,), jnp.int32),
            pltpu.VMEM((D // 8, T // 128, 8, 128), jnp.float32),
        ],
    )
    def k(cb_hbm, idx_hbm, out_hbm, cb_v, idx_v, zq_v):
        nc = lax.axis_size("c")
        wid = lax.axis_index("s") * nc + lax.axis_index("c")
        pltpu.sync_copy(cb_hbm, cb_v)
        pltpu.sync_copy(idx_hbm.at[wid], idx_v)

        @plsc.parallel_loop(0, T // 16, step=1, unroll=4)
        def band(i):
            ids16 = idx_v[pl.ds(i * 16, 16)]
            tb = i // 8
            lo = (i % 8) * 16
            for d in range(D):
                dvec = jnp.full((16,), d, jnp.int32)
                vals = plsc.load_gather(cb_v, [dvec, ids16])
                zq_v[d // 8, tb, d % 8, pl.ds(lo, 16)] = vals
        pltpu.sync_copy(zq_v, out_hbm.at[wid])

    return k(cbt, ids)


def kernel(z, codebook):
    B, T, D = z.shape
    K = codebook.shape[0]
    zt = jnp.swapaxes(z, 1, 2)
    cbt = codebook.T
    ids3, ids = _tc_ids(zt, cbt)
    zq5 = _sc_gather_t(cbt, ids3.reshape(B, T), B, D, T)
    zq_t = zq5.transpose(0, 1, 3, 2, 4).reshape(B, D, T)
    return jnp.swapaxes(zq_t, 1, 2), ids


# speedup vs baseline: 4.2003x; 1.0650x over previous
import functools

import jax
import jax.numpy as jnp
from jax import lax
from jax.experimental import pallas as pl
from jax.experimental.pallas import tpu as pltpu
from jax.experimental.pallas import tpu_sc as plsc


def _ids_body(zt_ref, cbt_ref, ids3_ref, ids_ref):
    cbt = cbt_ref[...]
    bias = 0.5 * jnp.sum(cbt * cbt, axis=0)[:, None]
    n_b = zt_ref.shape[0]
    n_ch, CH = ids3_ref.shape[2], ids3_ref.shape[3]
    for b in range(n_b):
        zbt = zt_ref[b]
        score = lax.dot_general(
            cbt, zbt, (((0,), (0,)), ((), ())),
            preferred_element_type=jnp.float32)
        score = score - bias
        ids = jnp.argmax(score, axis=0).astype(jnp.int32)
        ids3_ref[b, 0] = ids.reshape(n_ch, CH)
        ids_ref[b, 0] = ids


def _tc_ids(zt, cbt):
    B, D, T = zt.shape
    K = cbt.shape[1]
    n_ch, CH = T // 128, 128
    NB = 8
    ids3, ids = pl.pallas_call(
        _ids_body,
        grid=(B // NB,),
        in_specs=[
            pl.BlockSpec((NB, D, T), lambda i: (i, 0, 0)),
            pl.BlockSpec((D, K), lambda i: (0, 0)),
        ],
        out_specs=[
            pl.BlockSpec((NB, 1, n_ch, CH), lambda i: (i, 0, 0, 0)),
            pl.BlockSpec((NB, 1, T), lambda i: (i, 0, 0)),
        ],
        out_shape=[
            jax.ShapeDtypeStruct((B, 1, n_ch, CH), jnp.int32),
            jax.ShapeDtypeStruct((B, 1, T), jnp.int32),
        ],
    )(zt, cbt)
    return ids3.reshape(B, n_ch, CH), ids.reshape(B, T)


def _sc_gather_t(cbt, ids, B, D, T):
    K = cbt.shape[1]
    mesh = plsc.VectorSubcoreMesh(core_axis_name="c", subcore_axis_name="s")

    @functools.partial(
        pl.kernel,
        mesh=mesh,
        compiler_params=pltpu.CompilerParams(
            use_tc_tiling_on_sc=False, needs_layout_passes=False),
        out_type=jax.ShapeDtypeStruct((B, D // 8, T // 128, 8, 128),
                                      jnp.float32),
        scratch_types=[
            pltpu.VMEM((D, K), jnp.float32),
            pltpu.VMEM((T,), jnp.int32),
            pltpu.VMEM((D // 8, T // 128, 8, 128), jnp.float32),
            pltpu.SemaphoreType.DMA,
            pltpu.SemaphoreType.DMA,
            pltpu.SemaphoreType.DMA,
        ],
    )
    def k(cb_hbm, idx_hbm, out_hbm, cb_v, idx_v, zq_v, sem_cb, sem_ix,
          sem_out):
        nc = lax.axis_size("c")
        wid = lax.axis_index("s") * nc + lax.axis_index("c")
        cb_cp = pltpu.async_copy(cb_hbm, cb_v, sem_cb)
        ix_cp = pltpu.async_copy(idx_hbm.at[wid], idx_v, sem_ix)
        cb_cp.wait()
        ix_cp.wait()

        out_cps = []
        for p in range(D // 8):

            @plsc.parallel_loop(0, T // 16, step=1, unroll=8)
            def band(i, p=p):
                ids16 = idx_v[pl.ds(i * 16, 16)]
                tb = i // 8
                lo = (i % 8) * 16
                for dd in range(8):
                    dvec = jnp.full((16,), p * 8 + dd, jnp.int32)
                    vals = plsc.load_gather(cb_v, [dvec, ids16])
                    zq_v[p, tb, dd, pl.ds(lo, 16)] = vals

            out_cps.append(
                pltpu.async_copy(zq_v.at[p], out_hbm.at[wid, p], sem_out))
        for cp in out_cps:
            cp.wait()

    return k(cbt, ids)


def kernel(z, codebook):
    B, T, D = z.shape
    K = codebook.shape[0]
    zt = jnp.swapaxes(z, 1, 2)
    cbt = codebook.T
    ids3, ids = _tc_ids(zt, cbt)
    zq5 = _sc_gather_t(cbt, ids3.reshape(B, T), B, D, T)
    zq_t = zq5.transpose(0, 1, 3, 2, 4).reshape(B, D, T)
    return jnp.swapaxes(zq_t, 1, 2), ids


# speedup vs baseline: 4.2956x; 1.0227x over previous
import functools

import jax
import jax.numpy as jnp
from jax import lax
from jax.experimental import pallas as pl
from jax.experimental.pallas import tpu as pltpu
from jax.experimental.pallas import tpu_sc as plsc


def _ids_body(zt_ref, cbt_ref, ids3_ref):
    cbt = cbt_ref[...]
    bias = 0.5 * jnp.sum(cbt * cbt, axis=0)[:, None]
    n_b = zt_ref.shape[0]
    n_ch, CH = ids3_ref.shape[2], ids3_ref.shape[3]
    for b in range(n_b):
        zbt = zt_ref[b]
        score = lax.dot_general(
            cbt, zbt, (((0,), (0,)), ((), ())),
            preferred_element_type=jnp.float32)
        score = score - bias
        ids = jnp.argmax(score, axis=0).astype(jnp.int32)
        ids3_ref[b, 0] = ids.reshape(n_ch, CH)


def _tc_ids(zt, cbt):
    B, D, T = zt.shape
    K = cbt.shape[1]
    n_ch, CH = T // 128, 128
    NB = 8
    ids3 = pl.pallas_call(
        _ids_body,
        grid=(B // NB,),
        in_specs=[
            pl.BlockSpec((NB, D, T), lambda i: (i, 0, 0)),
            pl.BlockSpec((D, K), lambda i: (0, 0)),
        ],
        out_specs=pl.BlockSpec((NB, 1, n_ch, CH), lambda i: (i, 0, 0, 0)),
        out_shape=jax.ShapeDtypeStruct((B, 1, n_ch, CH), jnp.int32),
    )(zt, cbt)
    ids3 = ids3.reshape(B, n_ch, CH)
    ids = (ids3.reshape(B // 8, 8, n_ch, CH)
           .transpose(0, 2, 1, 3)
           .reshape(B, T))
    return ids3, ids


def _sc_gather_t(cbt, ids, B, D, T):
    K = cbt.shape[1]
    mesh = plsc.VectorSubcoreMesh(core_axis_name="c", subcore_axis_name="s")

    @functools.partial(
        pl.kernel,
        mesh=mesh,
        compiler_params=pltpu.CompilerParams(
            use_tc_tiling_on_sc=False, needs_layout_passes=False),
        out_type=jax.ShapeDtypeStruct((B, D // 8, T // 128, 8, 128),
                                      jnp.float32),
        scratch_types=[
            pltpu.VMEM((D, K), jnp.float32),
            pltpu.VMEM((T,), jnp.int32),
            pltpu.VMEM((D // 8, T // 128, 8, 128), jnp.float32),
            pltpu.SemaphoreType.DMA,
            pltpu.SemaphoreType.DMA,
            pltpu.SemaphoreType.DMA,
        ],
    )
    def k(cb_hbm, idx_hbm, out_hbm, cb_v, idx_v, zq_v, sem_cb, sem_ix,
          sem_out):
        nc = lax.axis_size("c")
        wid = lax.axis_index("s") * nc + lax.axis_index("c")
        cb_cp = pltpu.async_copy(cb_hbm, cb_v, sem_cb)
        ix_cp = pltpu.async_copy(idx_hbm.at[wid], idx_v, sem_ix)
        cb_cp.wait()
        ix_cp.wait()

        out_cps = []
        for p in range(D // 8):

            @plsc.parallel_loop(0, T // 16, step=1, unroll=8)
            def band(i, p=p):
                ids16 = idx_v[pl.ds(i * 16, 16)]
                tb = i // 8
                lo = (i % 8) * 16
                for dd in range(8):
                    dvec = jnp.full((16,), p * 8 + dd, jnp.int32)
                    vals = plsc.load_gather(cb_v, [dvec, ids16])
                    zq_v[p, tb, dd, pl.ds(lo, 16)] = vals

            out_cps.append(
                pltpu.async_copy(zq_v.at[p], out_hbm.at[wid, p], sem_out))
        for cp in out_cps:
            cp.wait()

    return k(cbt, ids)


def kernel(z, codebook):
    B, T, D = z.shape
    K = codebook.shape[0]
    zt = jnp.swapaxes(z, 1, 2)
    cbt = codebook.T
    ids3, ids = _tc_ids(zt, cbt)
    zq5 = _sc_gather_t(cbt, ids3.reshape(B, T), B, D, T)
    zq_t = zq5.transpose(0, 1, 3, 2, 4).reshape(B, D, T)
    return jnp.swapaxes(zq_t, 1, 2), ids
